# baseline (speedup 1.0000x reference)
import functools

import numpy as np
import jax
import jax.numpy as jnp
from jax import lax
from jax.experimental import pallas as pl
from jax.experimental.pallas import tpu as pltpu
from jax.experimental.pallas import tpu_sc as plsc

N_VALUES = 16777216
N_TOKENS = 4096
NC = 2
NS = 16
L = 16
NW = NC * NS
PER_W = N_VALUES // NW
CHUNK = 8192
N_CHUNKS = PER_W // CHUNK
UNROLL = 8
GROUPS = CHUNK // (UNROLL * L)
LEVELS = 12
EYT = N_TOKENS

_src = np.zeros(EYT, np.int32)
for _k in range(1, LEVELS + 1):
    _b = N_TOKENS >> _k
    _n = 1 << (_k - 1)
    _q = np.arange(_n, dtype=np.int32)
    _src[_n + _q] = _q * 2 * _b + _b - 1
_SRC = _src

_mesh = plsc.VectorSubcoreMesh(
    core_axis_name="c", subcore_axis_name="s",
    num_cores=NC, num_subcores=NS)


@functools.partial(
    pl.kernel,
    out_type=jax.ShapeDtypeStruct((N_VALUES,), jnp.int32),
    mesh=_mesh,
    compiler_params=pltpu.CompilerParams(needs_layout_passes=False),
    scratch_types=[
        pltpu.VMEM((EYT * L,), jnp.float32),
        pltpu.VMEM((CHUNK,), jnp.float32),
        pltpu.VMEM((CHUNK,), jnp.float32),
        pltpu.VMEM((CHUNK,), jnp.int32),
        pltpu.VMEM((CHUNK,), jnp.int32),
        pltpu.SemaphoreType.DMA,
        pltpu.SemaphoreType.DMA,
        pltpu.SemaphoreType.DMA,
        pltpu.SemaphoreType.DMA,
    ],
)
def _cdf_bin(inp_hbm, eyt_hbm, out_hbm, e_v, in0, in1, out0, out1,
             si0, si1, so0, so1):
    wid = lax.axis_index("s") * NC + lax.axis_index("c")
    base = wid * PER_W
    iota = lax.iota(jnp.int32, L)
    d_hi = L - iota
    d_lo = -iota
    s2_hi = iota + 3 * L
    s2_lo = iota + 2 * L

    pltpu.sync_copy(eyt_hbm, e_v)
    root = e_v[pl.ds(L, L)]

    def compute(in_v, out_v):
        @plsc.parallel_loop(0, GROUPS, unroll=4)
        def grp(g):
            goff = g * (UNROLL * L)
            xs = [in_v[pl.ds(goff + u * L, L)] for u in range(UNROLL)]
            ss = [jnp.where(root <= x, s2_hi, s2_lo) for x in xs]
            for _ in range(LEVELS - 1):
                vals = [plsc.load_gather(e_v, [s]) for s in ss]
                ss = [s + s + jnp.where(v <= x, d_hi, d_lo)
                      for s, v, x in zip(ss, vals, xs)]
            for u in range(UNROLL):
                out_v[pl.ds(goff + u * L, L)] = (ss[u] >> 4) - N_TOKENS

    ins = (in0, in1)
    outs = (out0, out1)
    sis = (si0, si1)
    sos = (so0, so1)

    pltpu.async_copy(inp_hbm.at[pl.ds(base, CHUNK)], in0, si0)

    def chunk_body(cidx, carry):
        for k in range(2):
            c = 2 * cidx + k
            off = base + c * CHUNK
            nxt_off = off + CHUNK
            @pl.when(c + 1 < N_CHUNKS)
            def _():
                pltpu.async_copy(inp_hbm.at[pl.ds(nxt_off, CHUNK)],
                                 ins[1 - k], sis[1 - k])
            @pl.when(c >= 2)
            def _():
                pltpu.make_async_copy(outs[k],
                                      out_hbm.at[pl.ds(off, CHUNK)],
                                      sos[k]).wait()
            pltpu.make_async_copy(inp_hbm.at[pl.ds(off, CHUNK)],
                                  ins[k], sis[k]).wait()
            compute(ins[k], outs[k])
            pltpu.async_copy(outs[k], out_hbm.at[pl.ds(off, CHUNK)],
                             sos[k])
        return carry

    lax.fori_loop(0, N_CHUNKS // 2, chunk_body, 0)

    for k in range(2):
        off = base + (N_CHUNKS - 2 + k) * CHUNK
        pltpu.make_async_copy(outs[k], out_hbm.at[pl.ds(off, CHUNK)],
                              sos[k]).wait()


def kernel(input, token_values):
    mids = (token_values[:-1] + token_values[1:]) * jnp.float32(0.5)
    mids = jnp.concatenate([mids, mids[-1:]])
    eyt = mids[_SRC]
    eyt_rep = jnp.repeat(eyt, L)
    return _cdf_bin(input, eyt_rep)


# speedup vs baseline: 1.0455x; 1.0455x over previous
import functools

import numpy as np
import jax
import jax.numpy as jnp
from jax import lax
from jax.experimental import pallas as pl
from jax.experimental.pallas import tpu as pltpu
from jax.experimental.pallas import tpu_sc as plsc

N_VALUES = 16777216
N_TOKENS = 4096
NC = 2
NS = 16
L = 16
NW = NC * NS
PER_W = N_VALUES // NW
CHUNK = 8192
N_CHUNKS = PER_W // CHUNK
UNROLL = 4
GROUPS = CHUNK // (UNROLL * L)
LEVELS = 12
EYT = N_TOKENS

_src = np.zeros(EYT, np.int32)
for _k in range(1, LEVELS + 1):
    _b = N_TOKENS >> _k
    _n = 1 << (_k - 1)
    _q = np.arange(_n, dtype=np.int32)
    _src[_n + _q] = _q * 2 * _b + _b - 1
_SRC = _src

_mesh = plsc.VectorSubcoreMesh(
    core_axis_name="c", subcore_axis_name="s",
    num_cores=NC, num_subcores=NS)


@functools.partial(
    pl.kernel,
    out_type=jax.ShapeDtypeStruct((N_VALUES,), jnp.int32),
    mesh=_mesh,
    compiler_params=pltpu.CompilerParams(needs_layout_passes=False),
    scratch_types=[
        pltpu.VMEM((EYT * L,), jnp.float32),
        pltpu.VMEM((CHUNK,), jnp.float32),
        pltpu.VMEM((CHUNK,), jnp.float32),
        pltpu.VMEM((CHUNK,), jnp.int32),
        pltpu.VMEM((CHUNK,), jnp.int32),
        pltpu.SemaphoreType.DMA,
        pltpu.SemaphoreType.DMA,
        pltpu.SemaphoreType.DMA,
        pltpu.SemaphoreType.DMA,
    ],
)
def _cdf_bin(inp_hbm, eyt_hbm, out_hbm, e_v, in0, in1, out0, out1,
             si0, si1, so0, so1):
    wid = lax.axis_index("s") * NC + lax.axis_index("c")
    base = wid * PER_W
    iota = lax.iota(jnp.int32, L)
    d_hi = L - iota
    d_lo = -iota
    s2_hi = iota + 3 * L
    s2_lo = iota + 2 * L

    pltpu.sync_copy(eyt_hbm, e_v)
    root = e_v[pl.ds(L, L)]

    def compute(in_v, out_v):
        @plsc.parallel_loop(0, GROUPS, unroll=4)
        def grp(g):
            goff = g * (UNROLL * L)
            xs = [in_v[pl.ds(goff + u * L, L)] for u in range(UNROLL)]
            ss = [jnp.where(root <= x, s2_hi, s2_lo) for x in xs]
            for _ in range(LEVELS - 1):
                vals = [plsc.load_gather(e_v, [s]) for s in ss]
                ss = [s + s + jnp.where(v <= x, d_hi, d_lo)
                      for s, v, x in zip(ss, vals, xs)]
            for u in range(UNROLL):
                out_v[pl.ds(goff + u * L, L)] = (ss[u] >> 4) - N_TOKENS

    ins = (in0, in1)
    outs = (out0, out1)
    sis = (si0, si1)
    sos = (so0, so1)

    pltpu.async_copy(inp_hbm.at[pl.ds(base, CHUNK)], in0, si0)

    def chunk_body(cidx, carry):
        for k in range(2):
            c = 2 * cidx + k
            off = base + c * CHUNK
            nxt_off = off + CHUNK
            @pl.when(c + 1 < N_CHUNKS)
            def _():
                pltpu.async_copy(inp_hbm.at[pl.ds(nxt_off, CHUNK)],
                                 ins[1 - k], sis[1 - k])
            @pl.when(c >= 2)
            def _():
                pltpu.make_async_copy(outs[k],
                                      out_hbm.at[pl.ds(off, CHUNK)],
                                      sos[k]).wait()
            pltpu.make_async_copy(inp_hbm.at[pl.ds(off, CHUNK)],
                                  ins[k], sis[k]).wait()
            compute(ins[k], outs[k])
            pltpu.async_copy(outs[k], out_hbm.at[pl.ds(off, CHUNK)],
                             sos[k])
        return carry

    lax.fori_loop(0, N_CHUNKS // 2, chunk_body, 0)

    for k in range(2):
        off = base + (N_CHUNKS - 2 + k) * CHUNK
        pltpu.make_async_copy(outs[k], out_hbm.at[pl.ds(off, CHUNK)],
                              sos[k]).wait()


def kernel(input, token_values):
    mids = (token_values[:-1] + token_values[1:]) * jnp.float32(0.5)
    mids = jnp.concatenate([mids, mids[-1:]])
    eyt = mids[_SRC]
    eyt_rep = jnp.repeat(eyt, L)
    return _cdf_bin(input, eyt_rep)
